# Initial kernel scaffold; baseline (speedup 1.0000x reference)
import jax
import jax.numpy as jnp
from jax.experimental import pallas as pl


def kernel(x, edge_index, all_edge_type, W, alpha_table, gamma, beta):
    raise NotImplementedError("write your pallas kernel here")


# speedup vs baseline: 16.9442x; 16.9442x over previous
import functools

import jax
import jax.numpy as jnp
from jax import lax
from jax.experimental import pallas as pl
from jax.experimental.pallas import tpu as pltpu
from jax.experimental.pallas import tpu_sc as plsc

N = 10000
D = 128
E = 320000
CHUNK = 128
NUM_CHUNKS = E // CHUNK
NC, NS = 2, 16
NW = NC * NS
ROWS_PER_TILE = 624
TAIL_ROWS = N - NS * ROWS_PER_TILE
ZROWS = 104
ALPHA_PAD = 224


def _sc_aggregate(x, src, dst, etype, alpha_sym):
    mesh = plsc.VectorSubcoreMesh(core_axis_name="c", subcore_axis_name="s")

    @functools.partial(
        pl.kernel,
        mesh=mesh,
        out_type=jax.ShapeDtypeStruct((NC, N, D), jnp.float32),
        scratch_types=[
            pltpu.VMEM((CHUNK,), jnp.int32),
            pltpu.VMEM((CHUNK,), jnp.int32),
            pltpu.VMEM((CHUNK,), jnp.int32),
            pltpu.VMEM((CHUNK, D), jnp.float32),
            pltpu.VMEM((ALPHA_PAD,), jnp.float32),
            pltpu.VMEM((ZROWS, D), jnp.float32),
            pltpu.VMEM_SHARED((N, D), jnp.float32),
            pltpu.SemaphoreType.DMA,
        ],
    )
    def k(x_hbm, src_hbm, dst_hbm, type_hbm, alpha_hbm, out_hbm,
          src_v, dst_v, type_v, rows_v, alpha_v, zero_v, acc_sh, sem):
        cid = lax.axis_index("c")
        sid = lax.axis_index("s")
        wid = sid * NC + cid

        pltpu.sync_copy(alpha_hbm, alpha_v)

        z16 = jnp.zeros((16,), jnp.float32)

        def zfill(i, _):
            zero_v[i // 8, pl.ds((i % 8) * 16, 16)] = z16
            return 0

        lax.fori_loop(0, ZROWS * 8, zfill, 0)
        base = sid * ROWS_PER_TILE

        def zcopy(i, _):
            pltpu.sync_copy(zero_v, acc_sh.at[pl.ds(base + i * ZROWS, ZROWS)])
            return 0

        lax.fori_loop(0, ROWS_PER_TILE // ZROWS, zcopy, 0)

        @pl.when(sid == NS - 1)
        def _zero_tail():
            pltpu.sync_copy(zero_v.at[pl.ds(0, TAIL_ROWS)],
                            acc_sh.at[pl.ds(NS * ROWS_PER_TILE, TAIL_ROWS)])

        plsc.subcore_barrier()

        nchunks = (NUM_CHUNKS - wid + NW - 1) // NW

        def chunk_body(ci, _):
            ebase = (wid + ci * NW) * CHUNK
            pltpu.sync_copy(src_hbm.at[pl.ds(ebase, CHUNK)], src_v)
            pltpu.sync_copy(dst_hbm.at[pl.ds(ebase, CHUNK)], dst_v)
            pltpu.sync_copy(type_hbm.at[pl.ds(ebase, CHUNK)], type_v)
            pltpu.async_copy(x_hbm.at[src_v], rows_v, sem).wait()

            def scale_grp(g, _):
                tv = type_v[pl.ds(g * 16, 16)]
                for k16 in range(16):
                    e = g * 16 + k16
                    av = alpha_v[pl.ds(tv[k16], 16)]
                    a_spl = jnp.full((16,), av[0], jnp.float32)
                    for cg in range(8):
                        sl = pl.ds(cg * 16, 16)
                        rows_v[e, sl] = rows_v[e, sl] * a_spl
                return 0

            lax.fori_loop(0, CHUNK // 16, scale_grp, 0)

            pltpu.sync_copy(rows_v, acc_sh.at[dst_v], add=True)
            return 0

        lax.fori_loop(0, nchunks, chunk_body, 0)
        plsc.subcore_barrier()

        pltpu.sync_copy(acc_sh.at[pl.ds(base, ROWS_PER_TILE)],
                        out_hbm.at[cid, pl.ds(base, ROWS_PER_TILE)])

        @pl.when(sid == NS - 1)
        def _drain_tail():
            pltpu.sync_copy(acc_sh.at[pl.ds(NS * ROWS_PER_TILE, TAIL_ROWS)],
                            out_hbm.at[cid, pl.ds(NS * ROWS_PER_TILE, TAIL_ROWS)])

    return k(x, src, dst, etype, alpha_sym)


def _tc_finish(partials, W, gamma, beta):
    def body(p_ref, w_ref, g_ref, b_ref, o_ref):
        agg = p_ref[0] + p_ref[1]
        feats = jnp.dot(agg, w_ref[...], preferred_element_type=jnp.float32)
        mean = jnp.mean(feats, axis=0, keepdims=True)
        dd = feats - mean
        var = jnp.mean(dd * dd, axis=0, keepdims=True)
        o_ref[...] = dd * lax.rsqrt(var + 1e-5) * g_ref[...] + b_ref[...]

    return pl.pallas_call(
        body,
        out_shape=jax.ShapeDtypeStruct((N, D), jnp.float32),
    )(partials, W, gamma.reshape(1, D), beta.reshape(1, D))


def kernel(x, edge_index, all_edge_type, W, alpha_table, gamma, beta):
    num_rel = alpha_table.shape[0]
    half = num_rel // 2
    table = alpha_table.at[0].set(0.0)[:, 0]
    r = jnp.arange(num_rel)
    transposed = jnp.where(r >= half, r - half, r + half)
    alpha_sym = jnp.pad(table + table[transposed], (0, ALPHA_PAD - num_rel))
    partials = _sc_aggregate(x, edge_index[0], edge_index[1], all_edge_type,
                             alpha_sym)
    return _tc_finish(partials, W, gamma, beta)
